# baseline (speedup 1.0000x reference)
import functools

import jax
import jax.numpy as jnp
from jax import lax
from jax.experimental import pallas as pl
from jax.experimental.pallas import tpu as pltpu
from jax.experimental.pallas import tpu_sc as plsc


@functools.lru_cache(maxsize=None)
def _build_gather(batch: int, nrows128: int):
    info = plsc.get_sparse_core_info()
    nw = info.num_cores * info.num_subcores
    lanes = info.num_lanes
    assert batch % (nw * lanes) == 0
    b_per_w = batch // nw
    nblk = b_per_w // lanes
    mesh = plsc.VectorSubcoreMesh(core_axis_name="c", subcore_axis_name="s")

    @functools.partial(
        pl.kernel,
        mesh=mesh,
        out_type=jax.ShapeDtypeStruct((batch * 16,), jnp.float32),
        scratch_types=[
            pltpu.VMEM((b_per_w,), jnp.int32),
            pltpu.VMEM((b_per_w,), jnp.int32),
            pltpu.VMEM((b_per_w, 128), jnp.float32),
            pltpu.VMEM((b_per_w * 16,), jnp.float32),
            pltpu.SemaphoreType.DMA,
        ],
        compiler_params=pltpu.CompilerParams(
            use_tc_tiling_on_sc=False, needs_layout_passes=False
        ),
    )
    def gather(idx_hbm, table_hbm, out_hbm, idx_v, row_v, rows_v, out_v, sem):
        wid = lax.axis_index("s") * info.num_cores + lax.axis_index("c")
        base = wid * b_per_w
        iota = lax.iota(jnp.int32, lanes)
        pltpu.sync_copy(idx_hbm.at[pl.ds(base, b_per_w)], idx_v)
        for t in range(nblk):
            v = idx_v[pl.ds(t * lanes, lanes)]
            row_v[pl.ds(t * lanes, lanes)] = v >> 3
        pltpu.async_copy(table_hbm.at[row_v], rows_v, sem).wait()
        for t in range(nblk):
            v = idx_v[pl.ds(t * lanes, lanes)]
            s = (v & 7) << 4
            src_row = t * lanes + iota
            dst0 = (t * lanes + iota) * 16
            for j in range(16):
                g = plsc.load_gather(rows_v, [src_row, s + j])
                plsc.store_scatter(out_v, [dst0 + j], g)
        pltpu.sync_copy(out_v, out_hbm.at[pl.ds(base * 16, b_per_w * 16)])

    return gather


def kernel(x, data):
    batch = x.shape[0]
    inter = x.shape[1:-1]
    idx = x.reshape(-1).astype(jnp.int32)
    table128 = data.reshape(-1, 128)
    out = _build_gather(idx.shape[0], table128.shape[0])(idx, table128)
    return out.reshape((batch,) + tuple(inter) + tuple(data.shape[1:]))


# speedup vs baseline: 3.5842x; 3.5842x over previous
import functools

import jax
import jax.numpy as jnp
from jax import lax
from jax.experimental import pallas as pl
from jax.experimental.pallas import tpu as pltpu
from jax.experimental.pallas import tpu_sc as plsc


@functools.lru_cache(maxsize=None)
def _build_gather(batch: int, nb_rows: int):
    info = plsc.get_sparse_core_info()
    nw = info.num_cores * info.num_subcores
    lanes = info.num_lanes
    assert batch % nw == 0
    b_per_w = batch // nw
    nslot = 4
    mesh = plsc.VectorSubcoreMesh(core_axis_name="c", subcore_axis_name="s")

    @functools.partial(
        pl.kernel,
        mesh=mesh,
        out_type=jax.ShapeDtypeStruct((batch * 16,), jnp.float32),
        scratch_types=[
            pltpu.VMEM((b_per_w + lanes,), jnp.int32),
            pltpu.VMEM((nslot, 2, 8, 128), jnp.float32),
            pltpu.VMEM((b_per_w * 16,), jnp.float32),
            [pltpu.SemaphoreType.DMA] * nslot,
        ],
        compiler_params=pltpu.CompilerParams(needs_layout_passes=False),
    )
    def gather(idx_hbm, table_hbm, out_hbm, idx_s, tiles_v, out_v, sems):
        wid = lax.axis_index("s") * info.num_cores + lax.axis_index("c")
        base = wid * b_per_w
        iota = lax.iota(jnp.int32, lanes)
        rr = iota & 7
        pltpu.sync_copy(idx_hbm.at[pl.ds(base, b_per_w)], idx_s.at[pl.ds(0, b_per_w)])

        def read_idx(item):
            return idx_s[pl.ds(item, lanes)][0]

        def issue(item, slot):
            c = read_idx(item)
            cb = pl.multiple_of((c >> 7) << 7, 128)
            for tr in range(2):
                pltpu.async_copy(
                    table_hbm.at[tr].at[:, pl.ds(cb, 128)],
                    tiles_v.at[slot, tr],
                    sems[slot],
                )

        def extract(item, slot):
            c = read_idx(item)
            coff = jnp.full((lanes,), c & 127, jnp.int32)
            g0 = plsc.load_gather(tiles_v.at[slot, 0], [rr, coff])
            g1 = plsc.load_gather(tiles_v.at[slot, 1], [rr, coff])
            out_v[pl.ds(item * 16, 16)] = jnp.where(iota < 8, g0, g1)

        for s in range(nslot):
            issue(s, s)

        def body(g, _):
            for s in range(nslot):
                item = g * nslot + s
                pltpu.make_async_copy(
                    table_hbm.at[0].at[:, pl.ds(0, 128)],
                    tiles_v.at[s, 0],
                    sems[s],
                ).wait()
                pltpu.make_async_copy(
                    table_hbm.at[0].at[:, pl.ds(0, 128)],
                    tiles_v.at[s, 1],
                    sems[s],
                ).wait()
                extract(item, s)

                @pl.when(item + nslot < b_per_w)
                def _():
                    issue(item + nslot, s)

            return 0

        lax.fori_loop(0, b_per_w // nslot, body, 0)
        pltpu.sync_copy(out_v, out_hbm.at[pl.ds(base * 16, b_per_w * 16)])

    return gather


def kernel(x, data):
    batch = x.shape[0]
    inter = x.shape[1:-1]
    idx = x.reshape(-1).astype(jnp.int32)
    table = data.T.reshape(2, 8, data.shape[0])
    out = _build_gather(idx.shape[0], data.shape[0])(idx, table)
    return out.reshape((batch,) + tuple(inter) + (data.shape[1],))


# speedup vs baseline: 4.6128x; 1.2870x over previous
import functools

import jax
import jax.numpy as jnp
from jax import lax
from jax.experimental import pallas as pl
from jax.experimental.pallas import tpu as pltpu
from jax.experimental.pallas import tpu_sc as plsc


@functools.lru_cache(maxsize=None)
def _build_gather(batch: int, nb_rows: int):
    info = plsc.get_sparse_core_info()
    nw = info.num_cores * info.num_subcores
    lanes = info.num_lanes
    assert batch % nw == 0
    b_per_w = batch // nw
    nslot = 16
    mesh = plsc.VectorSubcoreMesh(core_axis_name="c", subcore_axis_name="s")

    @functools.partial(
        pl.kernel,
        mesh=mesh,
        out_type=jax.ShapeDtypeStruct((batch * 16,), jnp.float32),
        scratch_types=[
            pltpu.VMEM((b_per_w + lanes,), jnp.int32),
            pltpu.VMEM((nslot, 2, 8, 128), jnp.float32),
            pltpu.VMEM((b_per_w * 16,), jnp.float32),
            [pltpu.SemaphoreType.DMA] * nslot,
        ],
        compiler_params=pltpu.CompilerParams(needs_layout_passes=False),
    )
    def gather(idx_hbm, table_hbm, out_hbm, idx_s, tiles_v, out_v, sems):
        wid = lax.axis_index("s") * info.num_cores + lax.axis_index("c")
        base = wid * b_per_w
        iota = lax.iota(jnp.int32, lanes)
        rr = iota & 7
        pltpu.sync_copy(idx_hbm.at[pl.ds(base, b_per_w)], idx_s.at[pl.ds(0, b_per_w)])

        def read_idx(item):
            return idx_s[pl.ds(item, lanes)][0]

        def issue(item, slot):
            c = read_idx(item)
            cb = pl.multiple_of((c >> 7) << 7, 128)
            for tr in range(2):
                pltpu.async_copy(
                    table_hbm.at[tr].at[:, pl.ds(cb, 128)],
                    tiles_v.at[slot, tr],
                    sems[slot],
                )

        def extract(item, slot):
            c = read_idx(item)
            coff = jnp.full((lanes,), c & 127, jnp.int32)
            g0 = plsc.load_gather(tiles_v.at[slot, 0], [rr, coff])
            g1 = plsc.load_gather(tiles_v.at[slot, 1], [rr, coff])
            out_v[pl.ds(item * 16, 16)] = jnp.where(iota < 8, g0, g1)

        for s in range(nslot):
            issue(s, s)

        def body(g, _):
            for s in range(nslot):
                item = g * nslot + s
                pltpu.make_async_copy(
                    table_hbm.at[0].at[:, pl.ds(0, 128)],
                    tiles_v.at[s, 0],
                    sems[s],
                ).wait()
                pltpu.make_async_copy(
                    table_hbm.at[0].at[:, pl.ds(0, 128)],
                    tiles_v.at[s, 1],
                    sems[s],
                ).wait()
                extract(item, s)

                @pl.when(item + nslot < b_per_w)
                def _():
                    issue(item + nslot, s)

            return 0

        lax.fori_loop(0, b_per_w // nslot, body, 0)
        pltpu.sync_copy(out_v, out_hbm.at[pl.ds(base * 16, b_per_w * 16)])

    return gather


def kernel(x, data):
    batch = x.shape[0]
    inter = x.shape[1:-1]
    idx = x.reshape(-1).astype(jnp.int32)
    table = data.T.reshape(2, 8, data.shape[0])
    out = _build_gather(idx.shape[0], data.shape[0])(idx, table)
    return out.reshape((batch,) + tuple(inter) + (data.shape[1],))
